# baseline (speedup 1.0000x reference)
import functools

import jax
import jax.numpy as jnp
from jax.experimental import pallas as pl
from jax.experimental.pallas import tpu as pltpu

_HIDDEN = 4096
_EXPERTS = 64
_TM = 512


def _gate_tile(x_ref, w_ref, scores_ref, idx_ref):
    x = x_ref[...]
    w = w_ref[...]
    logits = jax.lax.dot_general(
        x, w, (((1,), (1,)), ((), ())),
        preferred_element_type=jnp.float32)

    cols = jax.lax.broadcasted_iota(jnp.int32, logits.shape, 1)
    big = jnp.int32(_EXPERTS)

    m1 = jnp.max(logits, axis=1, keepdims=True)
    i1 = jnp.min(jnp.where(logits == m1, cols, big), axis=1, keepdims=True)
    masked = jnp.where(cols == i1, jnp.finfo(jnp.float32).min, logits)
    m2 = jnp.max(masked, axis=1, keepdims=True)
    i2 = jnp.min(jnp.where(masked == m2, cols, big), axis=1, keepdims=True)

    e2 = jnp.exp(m2 - m1)
    s1 = 1.0 / (1.0 + e2)
    scores_ref[...] = jnp.concatenate([s1, e2 * s1], axis=1)
    idx_ref[...] = jnp.concatenate([i1, i2], axis=1)


@functools.partial(jax.jit, static_argnames=())
def kernel(x, W):
    n_tokens = x.shape[0]
    grid = (n_tokens // _TM,)
    scores, idx = pl.pallas_call(
        _gate_tile,
        grid=grid,
        in_specs=[
            pl.BlockSpec((_TM, _HIDDEN), lambda i: (i, 0)),
            pl.BlockSpec((_EXPERTS, _HIDDEN), lambda i: (0, 0)),
        ],
        out_specs=[
            pl.BlockSpec((_TM, 2), lambda i: (i, 0)),
            pl.BlockSpec((_TM, 2), lambda i: (i, 0)),
        ],
        out_shape=[
            jax.ShapeDtypeStruct((n_tokens, 2), jnp.float32),
            jax.ShapeDtypeStruct((n_tokens, 2), jnp.int32),
        ],
        compiler_params=pltpu.CompilerParams(
            dimension_semantics=("parallel",)),
    )(x, W)
    return (scores, idx)


# speedup vs baseline: 1.0699x; 1.0699x over previous
import functools

import jax
import jax.numpy as jnp
from jax.experimental import pallas as pl
from jax.experimental.pallas import tpu as pltpu

_HIDDEN = 4096
_EXPERTS = 64
_TM = 1024


def _gate_tile(x_ref, w_ref, scores_ref, idx_ref):
    x = x_ref[...]
    w = w_ref[...]
    logits = jax.lax.dot_general(
        x, w, (((1,), (1,)), ((), ())),
        preferred_element_type=jnp.float32)

    cols = jax.lax.broadcasted_iota(jnp.int32, logits.shape, 1)
    big = jnp.int32(_EXPERTS)

    m1 = jnp.max(logits, axis=1, keepdims=True)
    i1 = jnp.min(jnp.where(logits == m1, cols, big), axis=1, keepdims=True)
    masked = jnp.where(cols == i1, jnp.finfo(jnp.float32).min, logits)
    m2 = jnp.max(masked, axis=1, keepdims=True)
    i2 = jnp.min(jnp.where(masked == m2, cols, big), axis=1, keepdims=True)

    e2 = jnp.exp(m2 - m1)
    s1 = 1.0 / (1.0 + e2)
    scores_ref[...] = jnp.concatenate([s1, e2 * s1], axis=1)
    idx_ref[...] = jnp.concatenate([i1, i2], axis=1)


@functools.partial(jax.jit, static_argnames=())
def kernel(x, W):
    n_tokens = x.shape[0]
    grid = (n_tokens // _TM,)
    scores, idx = pl.pallas_call(
        _gate_tile,
        grid=grid,
        in_specs=[
            pl.BlockSpec((_TM, _HIDDEN), lambda i: (i, 0)),
            pl.BlockSpec((_EXPERTS, _HIDDEN), lambda i: (0, 0)),
        ],
        out_specs=[
            pl.BlockSpec((_TM, 2), lambda i: (i, 0)),
            pl.BlockSpec((_TM, 2), lambda i: (i, 0)),
        ],
        out_shape=[
            jax.ShapeDtypeStruct((n_tokens, 2), jnp.float32),
            jax.ShapeDtypeStruct((n_tokens, 2), jnp.int32),
        ],
        compiler_params=pltpu.CompilerParams(
            dimension_semantics=("parallel",)),
    )(x, W)
    return (scores, idx)
